# Initial kernel scaffold; baseline (speedup 1.0000x reference)
import functools

import numpy as np
import jax
import jax.numpy as jnp
from jax import lax
from jax.experimental import pallas as pl
from jax.experimental.pallas import tpu as pltpu
from jax.experimental.pallas import tpu_sc as plsc

NUM_LEVELS = 16
LEVEL_DIM = 2
LOG2_T = 19
T = 1 << LOG2_T
BASE_RES = 16
PER_LEVEL_SCALE = 1.3819128800392151
N = 262144
OUT_DIM = NUM_LEVELS * LEVEL_DIM

_P1 = int(np.uint32(2654435761).astype(np.int32))
_P2 = 805459861
_MASK = T - 1

NC, NS = 2, 16
NW = NC * NS
PER_W = N // NW
C = 512
G = C // 16
NCH = PER_W // C

_RES = [float(np.floor(BASE_RES * PER_LEVEL_SCALE ** l)) for l in range(NUM_LEVELS)]

_mesh = plsc.VectorSubcoreMesh(core_axis_name="c", subcore_axis_name="s")


@functools.partial(
    pl.kernel,
    mesh=_mesh,
    out_type=jax.ShapeDtypeStruct((N, OUT_DIM), jnp.float32),
    scratch_types=[
        pltpu.VMEM((C, 3), jnp.float32),
        pltpu.VMEM((G, 128), jnp.int32),
        pltpu.VMEM((G, 128, LEVEL_DIM), jnp.float32),
        pltpu.VMEM((8, C), jnp.float32),
        pltpu.VMEM((C, OUT_DIM), jnp.float32),
        pltpu.SemaphoreType.DMA,
    ],
)
def _encode(coords_hbm, table_hbm, out_hbm, coords_v, idx_v, rows_v, w_v,
            out_v, sem):
    wid = lax.axis_index("s") * NC + lax.axis_index("c")
    iota = lax.iota(jnp.int32, 16)
    zeros_i = jnp.zeros((16,), jnp.int32)
    ones_i = zeros_i + 1
    twos_i = zeros_i + 2

    def chunk_body(ch, _):
        base = wid * PER_W + ch * C
        pltpu.sync_copy(coords_hbm.at[pl.ds(base, C)], coords_v)

        for l in range(NUM_LEVELS):
            res = _RES[l]
            loff = jnp.int32(l << LOG2_T)

            def hash_body(g, _, res=res, loff=loff):
                row = g * 16 + iota
                x = plsc.load_gather(coords_v, [row, zeros_i])
                y = plsc.load_gather(coords_v, [row, ones_i])
                z = plsc.load_gather(coords_v, [row, twos_i])
                px = x * res
                py = y * res
                pz = z * res
                ix = px.astype(jnp.int32)
                iy = py.astype(jnp.int32)
                iz = pz.astype(jnp.int32)
                fx = px - ix.astype(jnp.float32)
                fy = py - iy.astype(jnp.float32)
                fz = pz - iz.astype(jnp.float32)
                hy0 = iy * _P1
                hy1 = hy0 + _P1
                hz0 = iz * _P2
                hz1 = hz0 + _P2
                hx1 = ix + 1
                txy = (ix ^ hy0, ix ^ hy1, hx1 ^ hy0, hx1 ^ hy1)
                hz = (hz0, hz1)
                gx0 = 1.0 - fx
                gy0 = 1.0 - fy
                gz0 = 1.0 - fz
                wxy = (gx0 * gy0, gx0 * fy, fx * gy0, fx * fy)
                wz = (gz0, fz)
                for c in range(8):
                    i_, j_, k_ = c >> 2, (c >> 1) & 1, c & 1
                    idx = ((txy[i_ * 2 + j_] ^ hz[k_]) & _MASK) | loff
                    idx_v[g, pl.ds(c * 16, 16)] = idx
                    w_v[c, pl.ds(g * 16, 16)] = wxy[i_ * 2 + j_] * wz[k_]
                return 0

            lax.fori_loop(0, G, hash_body, 0)

            pltpu.async_copy(table_hbm.at[idx_v], rows_v, sem).wait()

            def interp_body(g, _, l=l):
                gs = zeros_i + g
                acc0 = jnp.zeros((16,), jnp.float32)
                acc1 = jnp.zeros((16,), jnp.float32)
                for c in range(8):
                    ridx = iota + c * 16
                    w = w_v[c, pl.ds(g * 16, 16)]
                    f0 = plsc.load_gather(rows_v, [gs, ridx, zeros_i])
                    f1 = plsc.load_gather(rows_v, [gs, ridx, ones_i])
                    acc0 = acc0 + f0 * w
                    acc1 = acc1 + f1 * w
                rowv = g * 16 + iota
                plsc.store_scatter(out_v, [rowv, zeros_i + (2 * l)], acc0)
                plsc.store_scatter(out_v, [rowv, zeros_i + (2 * l + 1)], acc1)
                return 0

            lax.fori_loop(0, G, interp_body, 0)

        pltpu.sync_copy(out_v, out_hbm.at[pl.ds(base, C)])
        return 0

    lax.fori_loop(0, NCH, chunk_body, 0)


def kernel(coords, table):
    table_flat = table.reshape(NUM_LEVELS * T, LEVEL_DIM)
    return _encode(coords, table_flat)


# speedup vs baseline: 21.4158x; 21.4158x over previous
import functools

import numpy as np
import jax
import jax.numpy as jnp
from jax import lax
from jax.experimental import pallas as pl
from jax.experimental.pallas import tpu as pltpu
from jax.experimental.pallas import tpu_sc as plsc

NUM_LEVELS = 16
LEVEL_DIM = 2
LOG2_T = 19
T = 1 << LOG2_T
BASE_RES = 16
PER_LEVEL_SCALE = 1.3819128800392151
N = 262144
OUT_DIM = NUM_LEVELS * LEVEL_DIM

_P1 = int(np.uint32(2654435761).astype(np.int32))
_P2 = 805459861
_MASK = T - 1

NC, NS = 2, 16
NW = NC * NS
PER_W = N // NW
C = 512
G = C // 16
NCH = PER_W // C

_RES = [float(np.floor(BASE_RES * PER_LEVEL_SCALE ** l)) for l in range(NUM_LEVELS)]

_mesh = plsc.VectorSubcoreMesh(core_axis_name="c", subcore_axis_name="s")


@functools.partial(
    pl.kernel,
    mesh=_mesh,
    compiler_params=pltpu.CompilerParams(
        needs_layout_passes=False, use_tc_tiling_on_sc=False),
    out_type=jax.ShapeDtypeStruct((N, OUT_DIM), jnp.float32),
    scratch_types=[
        pltpu.VMEM((C, 3), jnp.float32),
        pltpu.VMEM((8 * C,), jnp.int32),
        pltpu.VMEM((8 * C,), jnp.int32),
        pltpu.VMEM((8 * C, 8), jnp.float32),
        pltpu.VMEM((8, C), jnp.float32),
        pltpu.VMEM((C, OUT_DIM), jnp.float32),
        pltpu.SemaphoreType.DMA,
    ],
)
def _encode(coords_hbm, table_hbm, out_hbm, coords_v, idx_v, col_v, rows_v,
            w_v, out_v, sem):
    wid = lax.axis_index("s") * NC + lax.axis_index("c")
    iota = lax.iota(jnp.int32, 16)
    zeros_i = jnp.zeros((16,), jnp.int32)
    ones_i = zeros_i + 1
    twos_i = zeros_i + 2

    def chunk_body(ch, _):
        base = wid * PER_W + ch * C
        pltpu.sync_copy(coords_hbm.at[pl.ds(base, C)], coords_v)

        for l in range(NUM_LEVELS):
            res = _RES[l]
            loff = jnp.int32(l << LOG2_T)

            def hash_body(g, _, res=res, loff=loff):
                row = g * 16 + iota
                x = plsc.load_gather(coords_v, [row, zeros_i])
                y = plsc.load_gather(coords_v, [row, ones_i])
                z = plsc.load_gather(coords_v, [row, twos_i])
                px = x * res
                py = y * res
                pz = z * res
                ix = px.astype(jnp.int32)
                iy = py.astype(jnp.int32)
                iz = pz.astype(jnp.int32)
                fx = px - ix.astype(jnp.float32)
                fy = py - iy.astype(jnp.float32)
                fz = pz - iz.astype(jnp.float32)
                hy0 = iy * _P1
                hy1 = hy0 + _P1
                hz0 = iz * _P2
                hz1 = hz0 + _P2
                hx1 = ix + 1
                txy = (ix ^ hy0, ix ^ hy1, hx1 ^ hy0, hx1 ^ hy1)
                hz = (hz0, hz1)
                gx0 = 1.0 - fx
                gy0 = 1.0 - fy
                gz0 = 1.0 - fz
                wxy = (gx0 * gy0, gx0 * fy, fx * gy0, fx * fy)
                wz = (gz0, fz)
                for c in range(8):
                    i_, j_, k_ = c >> 2, (c >> 1) & 1, c & 1
                    idx = ((txy[i_ * 2 + j_] ^ hz[k_]) & _MASK) | loff
                    idx_v[pl.ds(g * 128 + c * 16, 16)] = idx >> 2
                    col_v[pl.ds(g * 128 + c * 16, 16)] = (idx & 3) << 1
                    w_v[c, pl.ds(g * 16, 16)] = wxy[i_ * 2 + j_] * wz[k_]
                return 0

            lax.fori_loop(0, G, hash_body, 0)

            pltpu.async_copy(table_hbm.at[idx_v], rows_v, sem).wait()

            def interp_body(g, _, l=l):
                gbase = g * 128 + iota
                acc0 = jnp.zeros((16,), jnp.float32)
                acc1 = jnp.zeros((16,), jnp.float32)
                for c in range(8):
                    ridx = gbase + c * 16
                    w = w_v[c, pl.ds(g * 16, 16)]
                    colv = col_v[pl.ds(g * 128 + c * 16, 16)]
                    f0 = plsc.load_gather(rows_v, [ridx, colv])
                    f1 = plsc.load_gather(rows_v, [ridx, colv + 1])
                    acc0 = acc0 + f0 * w
                    acc1 = acc1 + f1 * w
                rowv = g * 16 + iota
                plsc.store_scatter(out_v, [rowv, zeros_i + (2 * l)], acc0)
                plsc.store_scatter(out_v, [rowv, zeros_i + (2 * l + 1)], acc1)
                return 0

            lax.fori_loop(0, G, interp_body, 0)

        pltpu.sync_copy(out_v, out_hbm.at[pl.ds(base, C)])
        return 0

    lax.fori_loop(0, NCH, chunk_body, 0)


def kernel(coords, table):
    table_blk = table.reshape(NUM_LEVELS * T // 4, 4 * LEVEL_DIM)
    return _encode(coords, table_blk)


# speedup vs baseline: 86.2957x; 4.0295x over previous
import functools

import numpy as np
import jax
import jax.numpy as jnp
from jax import lax
from jax.experimental import pallas as pl
from jax.experimental.pallas import tpu as pltpu
from jax.experimental.pallas import tpu_sc as plsc

NUM_LEVELS = 16
LEVEL_DIM = 2
LOG2_T = 19
T = 1 << LOG2_T
BASE_RES = 16
PER_LEVEL_SCALE = 1.3819128800392151
N = 262144
OUT_DIM = NUM_LEVELS * LEVEL_DIM

_P1 = int(np.uint32(2654435761).astype(np.int32))
_P2 = 805459861
_MASK = T - 1

NC, NS = 2, 16
NW = NC * NS
PER_W = N // NW
C = 256
G = C // 16
NCH = PER_W // C

_RES = [float(np.floor(BASE_RES * PER_LEVEL_SCALE ** l)) for l in range(NUM_LEVELS)]

_mesh = plsc.VectorSubcoreMesh(core_axis_name="c", subcore_axis_name="s")

_BUF = lambda shape, dt: [pltpu.VMEM(shape, dt) for _ in range(2)]

@functools.partial(
    pl.kernel,
    mesh=_mesh,
    compiler_params=pltpu.CompilerParams(
        needs_layout_passes=False, use_tc_tiling_on_sc=False),
    out_type=jax.ShapeDtypeStruct((N, OUT_DIM), jnp.float32),
    scratch_types=[
        pltpu.VMEM((C, 3), jnp.float32),
        _BUF((8 * C,), jnp.int32),
        _BUF((8 * C,), jnp.int32),
        _BUF((8 * C,), jnp.int32),
        _BUF((8 * C, 8), jnp.float32),
        _BUF((8 * C, 8), jnp.float32),
        _BUF((8, C), jnp.float32),
        pltpu.VMEM((C, OUT_DIM), jnp.float32),
        [pltpu.SemaphoreType.DMA for _ in range(2)],
    ],
)
def _encode(coords_hbm, table_hbm, out_hbm, coords_v, idx0_b, idx1_b, col_b,
            rows0_b, rows1_b, w_b, out_v, sems):
    wid = lax.axis_index("s") * NC + lax.axis_index("c")
    iota = lax.iota(jnp.int32, 16)
    zeros_i = jnp.zeros((16,), jnp.int32)
    ones_i = zeros_i + 1
    twos_i = zeros_i + 2

    def hash_level(l, p):
        res = _RES[l]
        loffr = jnp.int32(l * (T // 4))
        idx0_v, idx1_v, col_v, w_v = idx0_b[p], idx1_b[p], col_b[p], w_b[p]

        def hash_body(g, _, res=res, loffr=loffr):
            row = g * 16 + iota
            x = plsc.load_gather(coords_v, [row, zeros_i])
            y = plsc.load_gather(coords_v, [row, ones_i])
            z = plsc.load_gather(coords_v, [row, twos_i])
            px = x * res
            py = y * res
            pz = z * res
            ix = px.astype(jnp.int32)
            iy = py.astype(jnp.int32)
            iz = pz.astype(jnp.int32)
            fx = px - ix.astype(jnp.float32)
            fy = py - iy.astype(jnp.float32)
            fz = pz - iz.astype(jnp.float32)
            hy0 = iy * _P1
            hy1 = hy0 + _P1
            hz0 = iz * _P2
            hz1 = hz0 + _P2
            hx1 = ix + 1
            txy = (ix ^ hy0, ix ^ hy1, hx1 ^ hy0, hx1 ^ hy1)
            hz = (hz0, hz1)
            gx0 = 1.0 - fx
            gy0 = 1.0 - fy
            gz0 = 1.0 - fz
            wxy = (gx0 * gy0, gx0 * fy, fx * gy0, fx * fy)
            wz = (gz0, fz)
            for c in range(8):
                i_, j_, k_ = c >> 2, (c >> 1) & 1, c & 1
                i = (txy[i_ * 2 + j_] ^ hz[k_]) & _MASK
                r0 = loffr + ((i >> 7) << 5) + ((i >> 3) & 15)
                idx0_v[pl.ds(g * 128 + c * 16, 16)] = r0
                idx1_v[pl.ds(g * 128 + c * 16, 16)] = r0 + 16
                col_v[pl.ds(g * 128 + c * 16, 16)] = i & 7
                w_v[c, pl.ds(g * 16, 16)] = wxy[i_ * 2 + j_] * wz[k_]
            return 0

        lax.fori_loop(0, G, hash_body, 0)

    def start_gather(p):
        cp0 = pltpu.async_copy(table_hbm.at[idx0_b[p]], rows0_b[p], sems[p])
        cp1 = pltpu.async_copy(table_hbm.at[idx1_b[p]], rows1_b[p], sems[p])
        return (cp0, cp1)

    def interp_level(l, p):
        col_v, rows0_v, rows1_v, w_v = col_b[p], rows0_b[p], rows1_b[p], w_b[p]

        def interp_body(g, _, l=l):
            gbase = g * 128 + iota
            acc0 = jnp.zeros((16,), jnp.float32)
            acc1 = jnp.zeros((16,), jnp.float32)
            for c in range(8):
                ridx = gbase + c * 16
                w = w_v[c, pl.ds(g * 16, 16)]
                colv = col_v[pl.ds(g * 128 + c * 16, 16)]
                f0 = plsc.load_gather(rows0_v, [ridx, colv])
                f1 = plsc.load_gather(rows1_v, [ridx, colv])
                acc0 = acc0 + f0 * w
                acc1 = acc1 + f1 * w
            rowv = g * 16 + iota
            plsc.store_scatter(out_v, [rowv, zeros_i + (2 * l)], acc0)
            plsc.store_scatter(out_v, [rowv, zeros_i + (2 * l + 1)], acc1)
            return 0

        lax.fori_loop(0, G, interp_body, 0)

    def chunk_body(ch, _):
        base = wid * PER_W + ch * C
        pltpu.sync_copy(coords_hbm.at[pl.ds(base, C)], coords_v)

        hash_level(0, 0)
        pending = start_gather(0)
        for l in range(NUM_LEVELS):
            p = l & 1
            q = p ^ 1
            if l < NUM_LEVELS - 1:
                hash_level(l + 1, q)
            for cp in pending:
                cp.wait()
            if l < NUM_LEVELS - 1:
                pending = start_gather(q)
            interp_level(l, p)

        pltpu.sync_copy(out_v, out_hbm.at[pl.ds(base, C)])
        return 0

    lax.fori_loop(0, NCH, chunk_body, 0)


def kernel(coords, table):
    table_v8 = (table.reshape(NUM_LEVELS, T // 128, 128, LEVEL_DIM)
                .transpose(0, 1, 3, 2)
                .reshape(NUM_LEVELS * (T // 128) * LEVEL_DIM * 16, 8))
    return _encode(coords, table_v8)


# speedup vs baseline: 173.4274x; 2.0097x over previous
import functools

import numpy as np
import jax
import jax.numpy as jnp
from jax import lax
from jax.experimental import pallas as pl
from jax.experimental.pallas import tpu as pltpu
from jax.experimental.pallas import tpu_sc as plsc

NUM_LEVELS = 16
LEVEL_DIM = 2
LOG2_T = 19
T = 1 << LOG2_T
BASE_RES = 16
PER_LEVEL_SCALE = 1.3819128800392151
N = 262144
OUT_DIM = NUM_LEVELS * LEVEL_DIM
TBL_F32 = NUM_LEVELS * T * LEVEL_DIM

_P1 = int(np.uint32(2654435761).astype(np.int32))
_P2 = 805459861
_MASK = T - 1

NC, NS = 2, 16
NW = NC * NS
PER_W = N // NW
C = 512
G = C // 16
NCH = PER_W // C

_RES = [float(np.floor(BASE_RES * PER_LEVEL_SCALE ** l)) for l in range(NUM_LEVELS)]

_mesh = plsc.VectorSubcoreMesh(core_axis_name="c", subcore_axis_name="s")
_CP = pltpu.CompilerParams(needs_layout_passes=False, use_tc_tiling_on_sc=False)

SBC = 128
NSB = TBL_F32 // 256
SB_PER_W = NSB // NW
NSTAGE = SB_PER_W // SBC


@functools.partial(
    pl.kernel,
    mesh=_mesh,
    compiler_params=_CP,
    out_type=jax.ShapeDtypeStruct((TBL_F32,), jnp.float32),
    scratch_types=[
        pltpu.VMEM((SBC * 256,), jnp.float32),
        pltpu.VMEM((SBC * 256,), jnp.float32),
    ],
)
def _interleave(src_hbm, dst_hbm, in_v, out_v):
    wid = lax.axis_index("s") * NC + lax.axis_index("c")
    iota2 = lax.iota(jnp.int32, 16) * 2

    def stage_body(st, _):
        base = (wid * SB_PER_W + st * SBC) * 256
        pltpu.sync_copy(src_hbm.at[pl.ds(base, SBC * 256)], in_v)

        def sb_body(s, _):
            sb = s * 256
            for j in range(8):
                v0 = in_v[pl.ds(sb + j * 16, 16)]
                v1 = in_v[pl.ds(sb + 128 + j * 16, 16)]
                oidx = sb + j * 32 + iota2
                plsc.store_scatter(out_v, [oidx], v0)
                plsc.store_scatter(out_v, [oidx + 1], v1)
            return 0

        lax.fori_loop(0, SBC, sb_body, 0)
        pltpu.sync_copy(out_v, dst_hbm.at[pl.ds(base, SBC * 256)])
        return 0

    lax.fori_loop(0, NSTAGE, stage_body, 0)


_BUF = lambda shape, dt: [pltpu.VMEM(shape, dt) for _ in range(2)]


@functools.partial(
    pl.kernel,
    mesh=_mesh,
    compiler_params=_CP,
    out_type=jax.ShapeDtypeStruct((OUT_DIM // 8, N // 128, 8, 128), jnp.float32),
    scratch_types=[
        pltpu.VMEM((C, 3), jnp.float32),
        _BUF((8 * C,), jnp.int32),
        _BUF((8 * C,), jnp.int32),
        _BUF((8 * C, 8), jnp.float32),
        _BUF((8, C), jnp.float32),
        pltpu.VMEM((OUT_DIM, C // 128, 128), jnp.float32),
        [pltpu.SemaphoreType.DMA for _ in range(3)],
    ],
)
def _encode(coords_hbm, table_hbm, out_hbm, coords_v, idx_b, col_b,
            rows_b, w_b, out_v, sems):
    wid = lax.axis_index("s") * NC + lax.axis_index("c")
    iota = lax.iota(jnp.int32, 16)
    zeros_i = jnp.zeros((16,), jnp.int32)
    ones_i = zeros_i + 1
    twos_i = zeros_i + 2

    def hash_level(l, p):
        res = _RES[l]
        loffr = jnp.int32(l * (T // 4))
        idx_v, col_v, w_v = idx_b[p], col_b[p], w_b[p]

        def hash_body(g, _, res=res, loffr=loffr):
            row = g * 16 + iota
            x = plsc.load_gather(coords_v, [row, zeros_i])
            y = plsc.load_gather(coords_v, [row, ones_i])
            z = plsc.load_gather(coords_v, [row, twos_i])
            px = x * res
            py = y * res
            pz = z * res
            ix = px.astype(jnp.int32)
            iy = py.astype(jnp.int32)
            iz = pz.astype(jnp.int32)
            fx = px - ix.astype(jnp.float32)
            fy = py - iy.astype(jnp.float32)
            fz = pz - iz.astype(jnp.float32)
            hy0 = iy * _P1
            hy1 = hy0 + _P1
            hz0 = iz * _P2
            hz1 = hz0 + _P2
            hx1 = ix + 1
            txy = (ix ^ hy0, ix ^ hy1, hx1 ^ hy0, hx1 ^ hy1)
            hz = (hz0, hz1)
            gx0 = 1.0 - fx
            gy0 = 1.0 - fy
            gz0 = 1.0 - fz
            wxy = (gx0 * gy0, gx0 * fy, fx * gy0, fx * fy)
            wz = (gz0, fz)
            for c in range(8):
                i_, j_, k_ = c >> 2, (c >> 1) & 1, c & 1
                i = (txy[i_ * 2 + j_] ^ hz[k_]) & _MASK
                idx_v[pl.ds(g * 128 + c * 16, 16)] = loffr + (i >> 2)
                col_v[pl.ds(g * 128 + c * 16, 16)] = (i & 3) << 1
                w_v[c, pl.ds(g * 16, 16)] = wxy[i_ * 2 + j_] * wz[k_]
            return 0

        lax.fori_loop(0, G, hash_body, 0)

    def start_gather(p):
        return pltpu.async_copy(table_hbm.at[idx_b[p]], rows_b[p], sems[p])

    def interp_level(l, p):
        col_v, rows_v, w_v = col_b[p], rows_b[p], w_b[p]

        for k in range(C // 128):
            def interp_body(g2, _, l=l, k=k):
                g = k * 8 + g2
                gbase = g * 128 + iota
                acc0 = jnp.zeros((16,), jnp.float32)
                acc1 = jnp.zeros((16,), jnp.float32)
                for c in range(8):
                    ridx = gbase + c * 16
                    w = w_v[c, pl.ds(g * 16, 16)]
                    colv = col_v[pl.ds(g * 128 + c * 16, 16)]
                    f0 = plsc.load_gather(rows_v, [ridx, colv])
                    f1 = plsc.load_gather(rows_v, [ridx, colv + 1])
                    acc0 = acc0 + f0 * w
                    acc1 = acc1 + f1 * w
                out_v[2 * l, k, pl.ds(g2 * 16, 16)] = acc0
                out_v[2 * l + 1, k, pl.ds(g2 * 16, 16)] = acc1
                return 0

            lax.fori_loop(0, 8, interp_body, 0)

    def chunk_body(ch, _):
        base = wid * PER_W + ch * C
        pltpu.sync_copy(coords_hbm.at[pl.ds(base, C)], coords_v)

        hash_level(0, 0)
        pending = start_gather(0)
        for l in range(NUM_LEVELS):
            p = l & 1
            q = p ^ 1
            nxt = None
            if l < NUM_LEVELS - 1:
                hash_level(l + 1, q)
                nxt = start_gather(q)
            pending.wait()
            interp_level(l, p)
            pending = nxt

        nb0 = base // 128
        cps = [
            pltpu.async_copy(
                out_v.at[col],
                out_hbm.at[col // 8, pl.ds(nb0, C // 128), col % 8],
                sems[2])
            for col in range(OUT_DIM)
        ]
        for cp in cps:
            cp.wait()
        return 0

    lax.fori_loop(0, NCH, chunk_body, 0)


def kernel(coords, table):
    table_flat = (table.reshape(NUM_LEVELS, T // 128, 128, LEVEL_DIM)
                  .transpose(0, 1, 3, 2)
                  .reshape(TBL_F32))
    table_rm = _interleave(table_flat)
    out4 = _encode(coords, table_rm.reshape(TBL_F32 // 8, 8))
    return out4.transpose(1, 3, 0, 2).reshape(N, OUT_DIM)
